# baseline (speedup 1.0000x reference)
import functools

import jax
import jax.numpy as jnp
from jax import lax
from jax.experimental import pallas as pl

_HB = 16


def _crop_kernel(rb_ref, data_ref, out_ref):
    n = rb_ref.shape[1]
    hb, w, _ = data_ref.shape
    x1 = rb_ref[0:1, :].reshape(1, 1, n)
    y1 = rb_ref[1:2, :].reshape(1, 1, n)
    x2 = rb_ref[2:3, :].reshape(1, 1, n)
    y2 = rb_ref[3:4, :].reshape(1, 1, n)

    ww = lax.broadcasted_iota(jnp.int32, (1, w, 1), 1).astype(jnp.float32)
    colmask = (ww >= x1) & (ww <= x2)

    h0 = (pl.program_id(0) * hb).astype(jnp.float32)
    hh = h0 + lax.broadcasted_iota(jnp.int32, (hb, 1, 1), 0).astype(jnp.float32)
    rowmask = (hh >= y1) & (hh <= y2)

    out_ref[...] = jnp.where(rowmask & colmask, data_ref[...], 0.0)


@jax.jit
def kernel(data, rois):
    height, width, n = data.shape
    rb = rois.T
    grid = (height // _HB,)
    return pl.pallas_call(
        _crop_kernel,
        grid=grid,
        in_specs=[
            pl.BlockSpec((4, n), lambda i: (0, 0)),
            pl.BlockSpec((_HB, width, n), lambda i: (i, 0, 0)),
        ],
        out_specs=pl.BlockSpec((_HB, width, n), lambda i: (i, 0, 0)),
        out_shape=jax.ShapeDtypeStruct((height, width, n), data.dtype),
    )(rb, data)
